# baseline (speedup 1.0000x reference)
import functools

import jax
import jax.numpy as jnp
from jax import lax
from jax.experimental import pallas as pl
from jax.experimental.pallas import tpu as pltpu
from jax.experimental.pallas import tpu_sc as plsc

C = 100000
D = 128
B = 16384
L = 16
NC = 2
NS = 16
NW = NC * NS
RW = B // NW
CH = 128
NCH = RW // CH
CPAD = 100352
RT = B // NS
WCH = 16
G = 64
ZB = 784
WSH = B

_mesh = plsc.VectorSubcoreMesh(core_axis_name="c", subcore_axis_name="s",
                               num_cores=NC, num_subcores=NS)
_cparams = pltpu.CompilerParams(needs_layout_passes=False)

_i32 = jnp.int32
_f32 = jnp.float32


def _iota():
    return lax.iota(_i32, L)


def _take16(arr, idx):
    dnums = lax.GatherDimensionNumbers(
        offset_dims=(), collapsed_slice_dims=(0,), start_index_map=(0,))
    return lax.gather(arr, idx[:, None], dnums, slice_sizes=(1,),
                      mode=lax.GatherScatterMode.PROMISE_IN_BOUNDS)


def _bcast_row(ref, r):
    v = ref[pl.ds(r, L)]
    return _take16(v, jnp.zeros((L,), _i32))


def _prefix_sum(x):
    it = _iota()
    for s in (1, 2, 4, 8):
        sh = _take16(x, jnp.maximum(it - s, 0))
        x = x + jnp.where(it >= s, sh, 0)
    return x


def _compact2(ref_a, ref_b, off, vals_a, vals_b, mask):
    pref = _prefix_sum(mask.astype(_i32))
    pos = off + pref - 1
    plsc.store_scatter(ref_a, [pos], vals_a, mask=mask)
    plsc.store_scatter(ref_b, [pos], vals_b, mask=mask)
    return off + pref[L - 1]


def _pad_replicate(ref_a, ref_b, n, nr):
    last_a = _bcast_row(ref_a, jnp.maximum(n - 1, 0))
    last_b = _bcast_row(ref_b, jnp.maximum(n - 1, 0))
    def fill(p, carry):
        pos = p * L + _iota()
        m = (pos >= n) & (pos < nr)
        plsc.store_scatter(ref_a, [pos], last_a, mask=m)
        plsc.store_scatter(ref_b, [pos], last_b, mask=m)
        return carry
    lax.fori_loop(n // L, (nr + L - 1) // L, fill, 0)


def _compute_rows(muv, xv, rcpv, nrows):
    @plsc.parallel_loop(0, nrows, unroll=2)
    def row_body(r):
        dv = _bcast_row(rcpv, r)
        for g in range(D // L):
            sl = pl.ds(g * L, L)
            m = muv[r, sl]
            xx = xv[r, sl]
            muv[r, sl] = m + (xx - m) * dv


def _k1_body(x_hbm, y_hbm, mu_hbm, ck_hbm, nmu, nck,
             idx2, ckv0, ckv1, rcpv, ckp0, ckp1, mu0, mu1, x0, x1,
             sidx, sin0, sin1, sout0, sout1):
    cid = lax.axis_index("c")
    sid = lax.axis_index("s")
    base = (sid * NC + cid) * RW
    dsc = [pltpu.async_copy(y_hbm.at[pl.ds(base + j * CH, CH)], idx2.at[j],
                            sidx) for j in range(NCH)]
    for d in dsc:
        d.wait()
    mub, xb, ckb, ckpb = [mu0, mu1], [x0, x1], [ckv0, ckv1], [ckp0, ckp1]
    sin, sout = [sin0, sin1], [sout0, sout1]

    def fire(j):
        s = j % 2
        return (pltpu.async_copy(mu_hbm.at[idx2.at[j]], mub[s], sin[s]),
                pltpu.async_copy(x_hbm.at[pl.ds(base + j * CH, CH)], xb[s],
                                 sin[s]),
                pltpu.async_copy(ck_hbm.at[idx2.at[j]], ckb[s], sin[s]))

    pend = [None, None]
    ind = fire(0)
    for j in range(NCH):
        s = j % 2
        if j + 1 < NCH:
            if pend[1 - s] is not None:
                for d in pend[1 - s]:
                    d.wait()
                pend[1 - s] = None
            nxt = fire(j + 1)
        for d in ind:
            d.wait()
        for v in range(CH // L):
            sl = pl.ds(v * L, L)
            cc = ckb[s][sl] + 1.0
            ckpb[s][sl] = cc
            rcpv[sl] = 1.0 / cc
        _compute_rows(mub[s], xb[s], rcpv, CH)
        pend[s] = (pltpu.async_copy(mub[s], nmu.at[idx2.at[j]], sout[s]),
                   pltpu.async_copy(ckpb[s], nck.at[idx2.at[j]], sout[s]))
        if j + 1 < NCH:
            ind = nxt
    for po in pend:
        if po is not None:
            for d in po:
                d.wait()


def _k2_body(x_hbm, y_hbm, mu_hbm, ck_hbm, nmu,
             cnt_sh, dupi_sh, dupy_sh, wini_sh, winy_sh,
             zbuf, idx3, onesb, cntv, dstage_i, dstage_y,
             tag, wstage_i, wstage_y,
             wli, wly, muv2, xv2, ckv2, rcpv2, smem, sz, si):
    cid = lax.axis_index("c")
    sid = lax.axis_index("s")

    for i in range(18):
        smem[i] = 0
    zslice = CPAD // NS
    def zb(i, carry):
        zbuf[pl.ds(i * L, L)] = jnp.zeros((L,), _i32)
        return carry
    lax.fori_loop(0, ZB // L, zb, 0)
    zd = [pltpu.async_copy(zbuf, cnt_sh.at[pl.ds(sid * zslice + q * ZB, ZB)],
                           sz) for q in range(zslice // ZB)]
    ld = [pltpu.async_copy(y_hbm.at[pl.ds(sid * RT + j * CH, CH)], idx3.at[j],
                           si) for j in range(RT // CH)]
    for v in range(CH // L):
        onesb[pl.ds(v * L, L)] = jnp.ones((L,), _i32)
    for d in zd + ld:
        d.wait()
    plsc.subcore_barrier()

    hd = [pltpu.async_copy(onesb, cnt_sh.at[idx3.at[j]], sz, add=True)
          for j in range(RT // CH)]
    for d in hd:
        d.wait()
    plsc.subcore_barrier()

    gd = [pltpu.async_copy(cnt_sh.at[idx3.at[j]],
                           cntv.at[pl.ds(j * CH, CH)], si)
          for j in range(RT // CH)]
    for d in gd:
        d.wait()
    ndup = jnp.int32(0)
    for v in range(RT // L):
        cv = cntv[pl.ds(v * L, L)]
        yv = idx3[v // (CH // L), pl.ds((v % (CH // L)) * L, L)]
        iv = _iota() + (sid * RT + v * L)
        ndup = _compact2(dstage_i, dstage_y, ndup, iv, yv, cv > 1)
    nr = lax.bitwise_and(ndup + (G - 1), ~(G - 1))
    _pad_replicate(dstage_i, dstage_y, ndup, nr)
    def pub(tt, carry):
        plsc.fetch_and_add(smem.at[sid], nr, subcore_id=tt)
        return carry
    lax.fori_loop(0, NS, pub, 0)
    plsc.subcore_barrier()
    off = jnp.int32(0)
    for t in range(NS):
        off = off + jnp.where(t < sid, smem[t], 0)
    def cpk(k, carry):
        ko = pl.multiple_of(off + k * G, G)
        pltpu.sync_copy(dstage_i.at[pl.ds(pl.multiple_of(k * G, G), G)],
                        dupi_sh.at[pl.ds(ko, G)])
        pltpu.sync_copy(dstage_y.at[pl.ds(pl.multiple_of(k * G, G), G)],
                        dupy_sh.at[pl.ds(ko, G)])
        return carry
    lax.fori_loop(0, nr // G, cpk, 0)
    plsc.subcore_barrier()

    @pl.when(sid == 0)
    def _resolve():
        ptot = jnp.int32(0)
        for t in range(NS):
            ptot = ptot + smem[t]
        nchunks = (ptot + RT - 1) // RT
        def p1(c, carry):
            nv = jnp.minimum(RT, ptot - c * RT) // L
            pltpu.sync_copy(dupy_sh.at[pl.ds(pl.multiple_of(c * RT, RT), RT)],
                            dstage_y.at[pl.ds(0, RT)])
            def z1(v, carry2):
                yv = dstage_y[pl.ds(v * L, L)]
                plsc.store_scatter(tag, [yv], jnp.zeros((L,), _i32))
                return carry2
            lax.fori_loop(0, nv, z1, 0)
            return carry
        lax.fori_loop(0, nchunks, p1, 0)
        def p2(c, carry):
            nv = jnp.minimum(RT, ptot - c * RT) // L
            co = pl.multiple_of(c * RT, RT)
            pltpu.sync_copy(dupy_sh.at[pl.ds(co, RT)],
                            dstage_y.at[pl.ds(0, RT)])
            pltpu.sync_copy(dupi_sh.at[pl.ds(co, RT)],
                            dstage_i.at[pl.ds(0, RT)])
            def s1(v, carry2):
                yv = dstage_y[pl.ds(v * L, L)]
                iv = dstage_i[pl.ds(v * L, L)]
                key = yv * B + iv
                ks, _unused = plsc.sort_key_val(key, key)
                ys = lax.shift_right_logical(ks, 14)
                ii = lax.bitwise_and(ks, B - 1)
                nxt = _take16(ys, jnp.minimum(_iota() + 1, L - 1))
                keep = (ys != nxt) | (_iota() == (L - 1))
                plsc.store_scatter(tag, [ys], ii + 1, mask=keep)
                return carry2
            lax.fori_loop(0, nv, s1, 0)
            return carry
        lax.fori_loop(0, nchunks, p2, 0)
        def p3(c, wtot):
            nv = jnp.minimum(RT, ptot - c * RT) // L
            co = pl.multiple_of(c * RT, RT)
            pltpu.sync_copy(dupy_sh.at[pl.ds(co, RT)],
                            dstage_y.at[pl.ds(0, RT)])
            pltpu.sync_copy(dupi_sh.at[pl.ds(co, RT)],
                            dstage_i.at[pl.ds(0, RT)])
            def w1(v, woff):
                yv = dstage_y[pl.ds(v * L, L)]
                iv = dstage_i[pl.ds(v * L, L)]
                g = plsc.load_gather(tag, [yv])
                winm = (g == iv + 1) & ((iv % 2) == cid)
                return _compact2(wstage_i, wstage_y, woff, iv, yv, winm)
            nw = lax.fori_loop(0, nv, w1, jnp.int32(0))
            nwr = lax.bitwise_and(nw + (G - 1), ~(G - 1))
            _pad_replicate(wstage_i, wstage_y, nw, nwr)
            def wcp(k, carry2):
                ko = pl.multiple_of(wtot + k * G, G)
                pltpu.sync_copy(wstage_i.at[pl.ds(pl.multiple_of(k * G, G), G)],
                                wini_sh.at[pl.ds(ko, G)])
                pltpu.sync_copy(wstage_y.at[pl.ds(pl.multiple_of(k * G, G), G)],
                                winy_sh.at[pl.ds(ko, G)])
                return carry2
            lax.fori_loop(0, nwr // G, wcp, 0)
            return wtot + nwr
        nwin = lax.fori_loop(0, nchunks, p3, jnp.int32(0))
        def pubw(tt, carry):
            plsc.fetch_and_add(smem.at[16], nwin, subcore_id=tt)
            return carry
        lax.fori_loop(0, NS, pubw, 0)
    plsc.subcore_barrier()

    nwp = smem[16]
    nce = nwp // WCH
    trips = jnp.maximum((nce - sid + NS - 1) // NS, 0)
    def fix(m, carry):
        bw = pl.multiple_of((sid + m * NS) * WCH, WCH)
        pltpu.sync_copy(wini_sh.at[pl.ds(bw, WCH)], wli.at[0])
        pltpu.sync_copy(winy_sh.at[pl.ds(bw, WCH)], wly.at[0])
        pltpu.sync_copy(mu_hbm.at[wly.at[0]], muv2)
        pltpu.sync_copy(x_hbm.at[wli.at[0]], xv2)
        pltpu.sync_copy(ck_hbm.at[wly.at[0]], ckv2)
        for v in range(WCH // L):
            sl = pl.ds(v * L, L)
            rcpv2[sl] = 1.0 / (ckv2[sl] + 1.0)
        _compute_rows(muv2, xv2, rcpv2, WCH)
        pltpu.sync_copy(muv2, nmu.at[wly.at[0]])
        return carry
    lax.fori_loop(0, trips, fix, 0)


def _k3_body(y_hbm, nmu, out_hbm, idx2, b0, b1, sidx, sg0, sg1, sw0, sw1):
    cid = lax.axis_index("c")
    sid = lax.axis_index("s")
    base = (sid * NC + cid) * RW
    dsc = [pltpu.async_copy(y_hbm.at[pl.ds(base + j * CH, CH)], idx2.at[j],
                            sidx) for j in range(NCH)]
    for d in dsc:
        d.wait()
    bufs, sg, sw = [b0, b1], [sg0, sg1], [sw0, sw1]
    pend = [None, None]
    gcur = pltpu.async_copy(nmu.at[idx2.at[0]], bufs[0], sg[0])
    for j in range(NCH):
        s = j % 2
        if j + 1 < NCH:
            if pend[1 - s] is not None:
                pend[1 - s].wait()
                pend[1 - s] = None
            gnxt = pltpu.async_copy(nmu.at[idx2.at[j + 1]], bufs[1 - s],
                                    sg[1 - s])
        gcur.wait()
        pend[s] = pltpu.async_copy(bufs[s],
                                   out_hbm.at[pl.ds(base + j * CH, CH)],
                                   sw[s])
        if j + 1 < NCH:
            gcur = gnxt
    for po in pend:
        if po is not None:
            po.wait()


_k1 = functools.partial(
    pl.kernel, mesh=_mesh, out_type=(), compiler_params=_cparams,
    scratch_types=[
        pltpu.VMEM((NCH, CH), _i32),
        pltpu.VMEM((CH,), _f32),
        pltpu.VMEM((CH,), _f32),
        pltpu.VMEM((CH + L,), _f32),
        pltpu.VMEM((CH,), _f32),
        pltpu.VMEM((CH,), _f32),
        pltpu.VMEM((CH, D), _f32),
        pltpu.VMEM((CH, D), _f32),
        pltpu.VMEM((CH, D), _f32),
        pltpu.VMEM((CH, D), _f32),
        pltpu.SemaphoreType.DMA,
        pltpu.SemaphoreType.DMA,
        pltpu.SemaphoreType.DMA,
        pltpu.SemaphoreType.DMA,
        pltpu.SemaphoreType.DMA,
    ],
)(_k1_body)

_k2 = functools.partial(
    pl.kernel, mesh=_mesh, out_type=(), compiler_params=_cparams,
    scratch_types=[
        pltpu.VMEM_SHARED((CPAD,), _i32),
        pltpu.VMEM_SHARED((B,), _i32),
        pltpu.VMEM_SHARED((B,), _i32),
        pltpu.VMEM_SHARED((WSH,), _i32),
        pltpu.VMEM_SHARED((WSH,), _i32),
        pltpu.VMEM((ZB,), _i32),
        pltpu.VMEM((RT // CH, CH), _i32),
        pltpu.VMEM((CH,), _i32),
        pltpu.VMEM((RT,), _i32),
        pltpu.VMEM((RT + G,), _i32),
        pltpu.VMEM((RT + G,), _i32),
        pltpu.VMEM((CPAD,), _i32),
        pltpu.VMEM((RT + G,), _i32),
        pltpu.VMEM((RT + G,), _i32),
        pltpu.VMEM((1, WCH), _i32),
        pltpu.VMEM((1, WCH), _i32),
        pltpu.VMEM((WCH, D), _f32),
        pltpu.VMEM((WCH, D), _f32),
        pltpu.VMEM((WCH,), _f32),
        pltpu.VMEM((WCH + L,), _f32),
        pltpu.SMEM((32,), _i32),
        pltpu.SemaphoreType.DMA,
        pltpu.SemaphoreType.DMA,
    ],
)(_k2_body)

_k3 = functools.partial(
    pl.kernel, mesh=_mesh, compiler_params=_cparams,
    out_type=jax.ShapeDtypeStruct((B, D), _f32),
    scratch_types=[
        pltpu.VMEM((NCH, CH), _i32),
        pltpu.VMEM((CH, D), _f32),
        pltpu.VMEM((CH, D), _f32),
        pltpu.SemaphoreType.DMA,
        pltpu.SemaphoreType.DMA,
        pltpu.SemaphoreType.DMA,
        pltpu.SemaphoreType.DMA,
        pltpu.SemaphoreType.DMA,
    ],
)(_k3_body)


def kernel(x, y, muK, cK):
    nmu = jax.new_ref(muK)
    nck = jax.new_ref(cK)
    _k1(x, y, muK, cK, nmu, nck)
    _k2(x, y, muK, cK, nmu)
    out = _k3(y, nmu)
    return out, nmu[...], nck[...]


# speedup vs baseline: 1.0086x; 1.0086x over previous
import functools

import jax
import jax.numpy as jnp
from jax import lax
from jax.experimental import pallas as pl
from jax.experimental.pallas import tpu as pltpu
from jax.experimental.pallas import tpu_sc as plsc

C = 100000
D = 128
B = 16384
L = 16
NC = 2
NS = 16
NW = NC * NS
RW = B // NW
CH = 128
NCH = RW // CH
CPAD = 100352
RT = B // NS
WCH = 16
G = 64
ZB = 784
WSH = B

_mesh = plsc.VectorSubcoreMesh(core_axis_name="c", subcore_axis_name="s",
                               num_cores=NC, num_subcores=NS)
_cparams = pltpu.CompilerParams(needs_layout_passes=False)

_i32 = jnp.int32
_f32 = jnp.float32


def _iota():
    return lax.iota(_i32, L)


def _take16(arr, idx):
    dnums = lax.GatherDimensionNumbers(
        offset_dims=(), collapsed_slice_dims=(0,), start_index_map=(0,))
    return lax.gather(arr, idx[:, None], dnums, slice_sizes=(1,),
                      mode=lax.GatherScatterMode.PROMISE_IN_BOUNDS)


def _bcast_row(ref, r):
    v = ref[pl.ds(r, L)]
    return _take16(v, jnp.zeros((L,), _i32))


def _prefix_sum(x):
    it = _iota()
    for s in (1, 2, 4, 8):
        sh = _take16(x, jnp.maximum(it - s, 0))
        x = x + jnp.where(it >= s, sh, 0)
    return x


def _compact2(ref_a, ref_b, off, vals_a, vals_b, mask):
    pref = _prefix_sum(mask.astype(_i32))
    pos = off + pref - 1
    plsc.store_scatter(ref_a, [pos], vals_a, mask=mask)
    plsc.store_scatter(ref_b, [pos], vals_b, mask=mask)
    return off + pref[L - 1]


def _pad_replicate(ref_a, ref_b, n, nr):
    last_a = _bcast_row(ref_a, jnp.maximum(n - 1, 0))
    last_b = _bcast_row(ref_b, jnp.maximum(n - 1, 0))
    def fill(p, carry):
        pos = p * L + _iota()
        m = (pos >= n) & (pos < nr)
        plsc.store_scatter(ref_a, [pos], last_a, mask=m)
        plsc.store_scatter(ref_b, [pos], last_b, mask=m)
        return carry
    lax.fori_loop(n // L, (nr + L - 1) // L, fill, 0)


def _compute_rows(muv, xv, rcpv, nrows):
    def row_body(r, carry):
        dv = _bcast_row(rcpv, r)
        for g in range(D // L):
            sl = pl.ds(g * L, L)
            m = muv[r, sl]
            xx = xv[r, sl]
            muv[r, sl] = m + (xx - m) * dv
        return carry
    lax.fori_loop(0, nrows, row_body, 0)


def _k1_body(x_hbm, y_hbm, mu_hbm, ck_hbm, nmu, nck,
             dupi_hbm, dupy_hbm, cnts_hbm,
             cnt_sh,
             idx2, ckv0, ckv1, rcpv, ckp0, ckp1, mu0, mu1, x0, x1,
             zbuf, idx3, onesb, cntv, dstage_i, dstage_y, cbuf, smem,
             sidx, sin0, sin1, sout0, sout1, sz, si):
    cid = lax.axis_index("c")
    sid = lax.axis_index("s")
    base = (sid * NC + cid) * RW

    for i in range(18):
        smem[i] = 0
    zslice = CPAD // NS
    def zb(i, carry):
        zbuf[pl.ds(i * L, L)] = jnp.zeros((L,), _i32)
        return carry
    lax.fori_loop(0, ZB // L, zb, 0)
    zd = [pltpu.async_copy(zbuf, cnt_sh.at[pl.ds(sid * zslice + q * ZB, ZB)],
                           sz) for q in range(zslice // ZB)]
    ld = [pltpu.async_copy(y_hbm.at[pl.ds(sid * RT + j * CH, CH)], idx3.at[j],
                           si) for j in range(RT // CH)]
    for v in range(CH // L):
        onesb[pl.ds(v * L, L)] = jnp.ones((L,), _i32)

    dsc = [pltpu.async_copy(y_hbm.at[pl.ds(base + j * CH, CH)], idx2.at[j],
                            sidx) for j in range(NCH)]
    for d in dsc:
        d.wait()
    mub, xb, ckb, ckpb = [mu0, mu1], [x0, x1], [ckv0, ckv1], [ckp0, ckp1]
    sin, sout = [sin0, sin1], [sout0, sout1]

    def fire(j):
        s = j % 2
        return (pltpu.async_copy(mu_hbm.at[idx2.at[j]], mub[s], sin[s]),
                pltpu.async_copy(x_hbm.at[pl.ds(base + j * CH, CH)], xb[s],
                                 sin[s]),
                pltpu.async_copy(ck_hbm.at[idx2.at[j]], ckb[s], sin[s]))

    pend = [None, None]
    ind = fire(0)
    for j in range(NCH):
        s = j % 2
        if j + 1 < NCH:
            if pend[1 - s] is not None:
                for d in pend[1 - s]:
                    d.wait()
                pend[1 - s] = None
            nxt = fire(j + 1)
        for d in ind:
            d.wait()
        for v in range(CH // L):
            sl = pl.ds(v * L, L)
            cc = ckb[s][sl] + 1.0
            ckpb[s][sl] = cc
            rcpv[sl] = 1.0 / cc
        _compute_rows(mub[s], xb[s], rcpv, CH)
        pend[s] = (pltpu.async_copy(mub[s], nmu.at[idx2.at[j]], sout[s]),
                   pltpu.async_copy(ckpb[s], nck.at[idx2.at[j]], sout[s]))
        if j + 1 < NCH:
            ind = nxt
    for po in pend:
        if po is not None:
            for d in po:
                d.wait()

    for d in zd + ld:
        d.wait()
    plsc.subcore_barrier()
    hd = [pltpu.async_copy(onesb, cnt_sh.at[idx3.at[j]], sz, add=True)
          for j in range(RT // CH)]
    for d in hd:
        d.wait()
    plsc.subcore_barrier()
    gd = [pltpu.async_copy(cnt_sh.at[idx3.at[j]],
                           cntv.at[pl.ds(j * CH, CH)], si)
          for j in range(RT // CH)]
    for d in gd:
        d.wait()
    ndup = jnp.int32(0)
    for v in range(RT // L):
        cv = cntv[pl.ds(v * L, L)]
        yv = idx3[v // (CH // L), pl.ds((v % (CH // L)) * L, L)]
        iv = _iota() + (sid * RT + v * L)
        ndup = _compact2(dstage_i, dstage_y, ndup, iv, yv, cv > 1)
    nr = lax.bitwise_and(ndup + (G - 1), ~(G - 1))
    _pad_replicate(dstage_i, dstage_y, ndup, nr)
    def pub(tt, carry):
        plsc.fetch_and_add(smem.at[sid], nr, subcore_id=tt)
        return carry
    lax.fori_loop(0, NS, pub, 0)
    plsc.subcore_barrier()
    off = jnp.int32(0)
    for t in range(NS):
        off = off + jnp.where(t < sid, smem[t], 0)

    @pl.when(cid == 0)
    def _emit():
        def cpk(k, carry):
            ko = pl.multiple_of(off + k * G, G)
            pltpu.sync_copy(dstage_i.at[pl.ds(pl.multiple_of(k * G, G), G)],
                            dupi_hbm.at[pl.ds(ko, G)])
            pltpu.sync_copy(dstage_y.at[pl.ds(pl.multiple_of(k * G, G), G)],
                            dupy_hbm.at[pl.ds(ko, G)])
            return carry
        lax.fori_loop(0, nr // G, cpk, 0)

    @pl.when((cid == 0) & (sid == 0))
    def _emit_cnts():
        for t in range(NS):
            cbuf[pl.ds(0, L)] = jnp.where(_iota() == t, smem[t],
                                          cbuf[pl.ds(0, L)])
        pltpu.sync_copy(cbuf, cnts_hbm)


def _k2_body(x_hbm, y_hbm, mu_hbm, ck_hbm, dupi_hbm, dupy_hbm, cnts_hbm, nmu,
             wini_sh, winy_sh,
             dstage_i, dstage_y, tag, wstage_i, wstage_y,
             wli, wly, muv2, xv2, ckv2, rcpv2, cbuf, smem):
    cid = lax.axis_index("c")
    sid = lax.axis_index("s")
    for i in range(18):
        smem[i] = 0
    plsc.subcore_barrier()

    @pl.when(sid == 0)
    def _resolve():
        pltpu.sync_copy(cnts_hbm, cbuf)
        ptot = _prefix_sum(cbuf[pl.ds(0, L)])[L - 1]
        nchunks = (ptot + RT - 1) // RT
        def p1(c, carry):
            nv = jnp.minimum(RT, ptot - c * RT) // L
            pltpu.sync_copy(dupy_hbm.at[pl.ds(pl.multiple_of(c * RT, RT), RT)],
                            dstage_y.at[pl.ds(0, RT)])
            def z1(v, carry2):
                yv = dstage_y[pl.ds(v * L, L)]
                plsc.store_scatter(tag, [yv], jnp.zeros((L,), _i32))
                return carry2
            lax.fori_loop(0, nv, z1, 0)
            return carry
        lax.fori_loop(0, nchunks, p1, 0)
        def p2(c, carry):
            nv = jnp.minimum(RT, ptot - c * RT) // L
            co = pl.multiple_of(c * RT, RT)
            pltpu.sync_copy(dupy_hbm.at[pl.ds(co, RT)],
                            dstage_y.at[pl.ds(0, RT)])
            pltpu.sync_copy(dupi_hbm.at[pl.ds(co, RT)],
                            dstage_i.at[pl.ds(0, RT)])
            def s1(v, carry2):
                yv = dstage_y[pl.ds(v * L, L)]
                iv = dstage_i[pl.ds(v * L, L)]
                key = yv * B + iv
                ks, _unused = plsc.sort_key_val(key, key)
                ys = lax.shift_right_logical(ks, 14)
                ii = lax.bitwise_and(ks, B - 1)
                nxt = _take16(ys, jnp.minimum(_iota() + 1, L - 1))
                keep = (ys != nxt) | (_iota() == (L - 1))
                plsc.store_scatter(tag, [ys], ii + 1, mask=keep)
                return carry2
            lax.fori_loop(0, nv, s1, 0)
            return carry
        lax.fori_loop(0, nchunks, p2, 0)
        def p3(c, wtot):
            nv = jnp.minimum(RT, ptot - c * RT) // L
            co = pl.multiple_of(c * RT, RT)
            pltpu.sync_copy(dupy_hbm.at[pl.ds(co, RT)],
                            dstage_y.at[pl.ds(0, RT)])
            pltpu.sync_copy(dupi_hbm.at[pl.ds(co, RT)],
                            dstage_i.at[pl.ds(0, RT)])
            def w1(v, woff):
                yv = dstage_y[pl.ds(v * L, L)]
                iv = dstage_i[pl.ds(v * L, L)]
                g = plsc.load_gather(tag, [yv])
                winm = (g == iv + 1) & ((iv % 2) == cid)
                return _compact2(wstage_i, wstage_y, woff, iv, yv, winm)
            nw = lax.fori_loop(0, nv, w1, jnp.int32(0))
            nwr = lax.bitwise_and(nw + (G - 1), ~(G - 1))
            _pad_replicate(wstage_i, wstage_y, nw, nwr)
            def wcp(k, carry2):
                ko = pl.multiple_of(wtot + k * G, G)
                pltpu.sync_copy(wstage_i.at[pl.ds(pl.multiple_of(k * G, G), G)],
                                wini_sh.at[pl.ds(ko, G)])
                pltpu.sync_copy(wstage_y.at[pl.ds(pl.multiple_of(k * G, G), G)],
                                winy_sh.at[pl.ds(ko, G)])
                return carry2
            lax.fori_loop(0, nwr // G, wcp, 0)
            return wtot + nwr
        nwin = lax.fori_loop(0, nchunks, p3, jnp.int32(0))
        def pubw(tt, carry):
            plsc.fetch_and_add(smem.at[16], nwin, subcore_id=tt)
            return carry
        lax.fori_loop(0, NS, pubw, 0)
    plsc.subcore_barrier()

    nwp = smem[16]
    nce = nwp // WCH
    trips = jnp.maximum((nce - sid + NS - 1) // NS, 0)
    def fix(m, carry):
        bw = pl.multiple_of((sid + m * NS) * WCH, WCH)
        pltpu.sync_copy(wini_sh.at[pl.ds(bw, WCH)], wli.at[0])
        pltpu.sync_copy(winy_sh.at[pl.ds(bw, WCH)], wly.at[0])
        pltpu.sync_copy(mu_hbm.at[wly.at[0]], muv2)
        pltpu.sync_copy(x_hbm.at[wli.at[0]], xv2)
        pltpu.sync_copy(ck_hbm.at[wly.at[0]], ckv2)
        for v in range(WCH // L):
            sl = pl.ds(v * L, L)
            rcpv2[sl] = 1.0 / (ckv2[sl] + 1.0)
        _compute_rows(muv2, xv2, rcpv2, WCH)
        pltpu.sync_copy(muv2, nmu.at[wly.at[0]])
        return carry
    lax.fori_loop(0, trips, fix, 0)


def _k3_body(y_hbm, nmu, out_hbm, idx2, b0, b1, sidx, sg0, sg1, sw0, sw1):
    cid = lax.axis_index("c")
    sid = lax.axis_index("s")
    base = (sid * NC + cid) * RW
    dsc = [pltpu.async_copy(y_hbm.at[pl.ds(base + j * CH, CH)], idx2.at[j],
                            sidx) for j in range(NCH)]
    for d in dsc:
        d.wait()
    bufs, sg, sw = [b0, b1], [sg0, sg1], [sw0, sw1]
    pend = [None, None]
    gcur = pltpu.async_copy(nmu.at[idx2.at[0]], bufs[0], sg[0])
    for j in range(NCH):
        s = j % 2
        if j + 1 < NCH:
            if pend[1 - s] is not None:
                pend[1 - s].wait()
                pend[1 - s] = None
            gnxt = pltpu.async_copy(nmu.at[idx2.at[j + 1]], bufs[1 - s],
                                    sg[1 - s])
        gcur.wait()
        pend[s] = pltpu.async_copy(bufs[s],
                                   out_hbm.at[pl.ds(base + j * CH, CH)],
                                   sw[s])
        if j + 1 < NCH:
            gcur = gnxt
    for po in pend:
        if po is not None:
            po.wait()


_k1 = functools.partial(
    pl.kernel, mesh=_mesh, compiler_params=_cparams,
    out_type=(jax.ShapeDtypeStruct((B,), _i32),
              jax.ShapeDtypeStruct((B,), _i32),
              jax.ShapeDtypeStruct((L,), _i32)),
    scratch_types=[
        pltpu.VMEM_SHARED((CPAD,), _i32),
        pltpu.VMEM((NCH, CH), _i32),
        pltpu.VMEM((CH,), _f32),
        pltpu.VMEM((CH,), _f32),
        pltpu.VMEM((CH + L,), _f32),
        pltpu.VMEM((CH,), _f32),
        pltpu.VMEM((CH,), _f32),
        pltpu.VMEM((CH, D), _f32),
        pltpu.VMEM((CH, D), _f32),
        pltpu.VMEM((CH, D), _f32),
        pltpu.VMEM((CH, D), _f32),
        pltpu.VMEM((ZB,), _i32),
        pltpu.VMEM((RT // CH, CH), _i32),
        pltpu.VMEM((CH,), _i32),
        pltpu.VMEM((RT,), _i32),
        pltpu.VMEM((RT + G,), _i32),
        pltpu.VMEM((RT + G,), _i32),
        pltpu.VMEM((L,), _i32),
        pltpu.SMEM((32,), _i32),
        pltpu.SemaphoreType.DMA,
        pltpu.SemaphoreType.DMA,
        pltpu.SemaphoreType.DMA,
        pltpu.SemaphoreType.DMA,
        pltpu.SemaphoreType.DMA,
        pltpu.SemaphoreType.DMA,
        pltpu.SemaphoreType.DMA,
    ],
)(_k1_body)

_k2 = functools.partial(
    pl.kernel, mesh=_mesh, out_type=(), compiler_params=_cparams,
    scratch_types=[
        pltpu.VMEM_SHARED((WSH,), _i32),
        pltpu.VMEM_SHARED((WSH,), _i32),
        pltpu.VMEM((RT + G,), _i32),
        pltpu.VMEM((RT + G,), _i32),
        pltpu.VMEM((CPAD,), _i32),
        pltpu.VMEM((RT + G,), _i32),
        pltpu.VMEM((RT + G,), _i32),
        pltpu.VMEM((1, WCH), _i32),
        pltpu.VMEM((1, WCH), _i32),
        pltpu.VMEM((WCH, D), _f32),
        pltpu.VMEM((WCH, D), _f32),
        pltpu.VMEM((WCH,), _f32),
        pltpu.VMEM((WCH + L,), _f32),
        pltpu.VMEM((L,), _i32),
        pltpu.SMEM((32,), _i32),
    ],
)(_k2_body)

_k3 = functools.partial(
    pl.kernel, mesh=_mesh, compiler_params=_cparams,
    out_type=jax.ShapeDtypeStruct((B, D), _f32),
    scratch_types=[
        pltpu.VMEM((NCH, CH), _i32),
        pltpu.VMEM((CH, D), _f32),
        pltpu.VMEM((CH, D), _f32),
        pltpu.SemaphoreType.DMA,
        pltpu.SemaphoreType.DMA,
        pltpu.SemaphoreType.DMA,
        pltpu.SemaphoreType.DMA,
        pltpu.SemaphoreType.DMA,
    ],
)(_k3_body)


def kernel(x, y, muK, cK):
    nmu = jax.new_ref(muK)
    nck = jax.new_ref(cK)
    dupi, dupy, cnts = _k1(x, y, muK, cK, nmu, nck)
    _k2(x, y, muK, cK, dupi, dupy, cnts, nmu)
    out = _k3(y, nmu)
    return out, nmu[...], nck[...]


# speedup vs baseline: 1.0800x; 1.0708x over previous
import functools

import jax
import jax.numpy as jnp
from jax import lax
from jax.experimental import pallas as pl
from jax.experimental.pallas import tpu as pltpu
from jax.experimental.pallas import tpu_sc as plsc

C = 100000
D = 128
B = 16384
L = 16
NC = 2
NS = 16
NW = NC * NS
RW = B // NW
CH = 128
NCH = RW // CH
CPAD = 100352
RT = B // NS
WCH = 64
G = 64
ZB = 784
WSH = B

_mesh = plsc.VectorSubcoreMesh(core_axis_name="c", subcore_axis_name="s",
                               num_cores=NC, num_subcores=NS)
_cparams = pltpu.CompilerParams(needs_layout_passes=False)

_i32 = jnp.int32
_f32 = jnp.float32


def _iota():
    return lax.iota(_i32, L)


def _take16(arr, idx):
    dnums = lax.GatherDimensionNumbers(
        offset_dims=(), collapsed_slice_dims=(0,), start_index_map=(0,))
    return lax.gather(arr, idx[:, None], dnums, slice_sizes=(1,),
                      mode=lax.GatherScatterMode.PROMISE_IN_BOUNDS)


def _bcast_row(ref, r):
    v = ref[pl.ds(r, L)]
    return _take16(v, jnp.zeros((L,), _i32))


def _prefix_sum(x):
    it = _iota()
    for s in (1, 2, 4, 8):
        sh = _take16(x, jnp.maximum(it - s, 0))
        x = x + jnp.where(it >= s, sh, 0)
    return x


def _compact2(ref_a, ref_b, off, vals_a, vals_b, mask):
    pref = _prefix_sum(mask.astype(_i32))
    pos = off + pref - 1
    plsc.store_scatter(ref_a, [pos], vals_a, mask=mask)
    plsc.store_scatter(ref_b, [pos], vals_b, mask=mask)
    return off + pref[L - 1]


def _pad_replicate(ref_a, ref_b, n, nr):
    last_a = _bcast_row(ref_a, jnp.maximum(n - 1, 0))
    last_b = _bcast_row(ref_b, jnp.maximum(n - 1, 0))
    def fill(p, carry):
        pos = p * L + _iota()
        m = (pos >= n) & (pos < nr)
        plsc.store_scatter(ref_a, [pos], last_a, mask=m)
        plsc.store_scatter(ref_b, [pos], last_b, mask=m)
        return carry
    lax.fori_loop(n // L, (nr + L - 1) // L, fill, 0)


def _compute_rows(muv, xv, rcpv, nrows):
    def row_body(r, carry):
        dv = _bcast_row(rcpv, r)
        for g in range(D // L):
            sl = pl.ds(g * L, L)
            m = muv[r, sl]
            xx = xv[r, sl]
            muv[r, sl] = m + (xx - m) * dv
        return carry
    lax.fori_loop(0, nrows, row_body, 0)


def _k1_body(x_hbm, y_hbm, mu_hbm, ck_hbm, nmu, nck,
             dupi_hbm, dupy_hbm, cnts_hbm,
             cnt_sh,
             idx2, ckv0, ckv1, rcpv, ckp0, ckp1, mu0, mu1, x0, x1,
             zbuf, idx3, onesb, cntv, dstage_i, dstage_y, cbuf, smem,
             sidx, sin0, sin1, sout0, sout1, sz, si):
    cid = lax.axis_index("c")
    sid = lax.axis_index("s")
    base = (sid * NC + cid) * RW

    for i in range(18):
        smem[i] = 0
    zslice = CPAD // NS
    def zb(i, carry):
        zbuf[pl.ds(i * L, L)] = jnp.zeros((L,), _i32)
        return carry
    lax.fori_loop(0, ZB // L, zb, 0)
    zd = [pltpu.async_copy(zbuf, cnt_sh.at[pl.ds(sid * zslice + q * ZB, ZB)],
                           sz) for q in range(zslice // ZB)]
    ld = [pltpu.async_copy(y_hbm.at[pl.ds(sid * RT + j * CH, CH)], idx3.at[j],
                           si) for j in range(RT // CH)]
    for v in range(CH // L):
        onesb[pl.ds(v * L, L)] = jnp.ones((L,), _i32)

    dsc = [pltpu.async_copy(y_hbm.at[pl.ds(base + j * CH, CH)], idx2.at[j],
                            sidx) for j in range(NCH)]
    for d in dsc:
        d.wait()
    mub, xb, ckb, ckpb = [mu0, mu1], [x0, x1], [ckv0, ckv1], [ckp0, ckp1]
    sin, sout = [sin0, sin1], [sout0, sout1]

    def fire(j):
        s = j % 2
        return (pltpu.async_copy(mu_hbm.at[idx2.at[j]], mub[s], sin[s]),
                pltpu.async_copy(x_hbm.at[pl.ds(base + j * CH, CH)], xb[s],
                                 sin[s]),
                pltpu.async_copy(ck_hbm.at[idx2.at[j]], ckb[s], sin[s]))

    pend = [None, None]
    ind = fire(0)
    for j in range(NCH):
        s = j % 2
        if j + 1 < NCH:
            if pend[1 - s] is not None:
                for d in pend[1 - s]:
                    d.wait()
                pend[1 - s] = None
            nxt = fire(j + 1)
        for d in ind:
            d.wait()
        for v in range(CH // L):
            sl = pl.ds(v * L, L)
            cc = ckb[s][sl] + 1.0
            ckpb[s][sl] = cc
            rcpv[sl] = 1.0 / cc
        _compute_rows(mub[s], xb[s], rcpv, CH)
        pend[s] = (pltpu.async_copy(mub[s], nmu.at[idx2.at[j]], sout[s]),
                   pltpu.async_copy(ckpb[s], nck.at[idx2.at[j]], sout[s]))
        if j + 1 < NCH:
            ind = nxt
    for po in pend:
        if po is not None:
            for d in po:
                d.wait()

    for d in zd + ld:
        d.wait()
    plsc.subcore_barrier()
    hd = [pltpu.async_copy(onesb, cnt_sh.at[idx3.at[j]], sz, add=True)
          for j in range(RT // CH)]
    for d in hd:
        d.wait()
    plsc.subcore_barrier()
    gd = [pltpu.async_copy(cnt_sh.at[idx3.at[j]],
                           cntv.at[pl.ds(j * CH, CH)], si)
          for j in range(RT // CH)]
    for d in gd:
        d.wait()
    ndup = jnp.int32(0)
    for v in range(RT // L):
        cv = cntv[pl.ds(v * L, L)]
        yv = idx3[v // (CH // L), pl.ds((v % (CH // L)) * L, L)]
        iv = _iota() + (sid * RT + v * L)
        ndup = _compact2(dstage_i, dstage_y, ndup, iv, yv, cv > 1)
    nr = lax.bitwise_and(ndup + (G - 1), ~(G - 1))
    _pad_replicate(dstage_i, dstage_y, ndup, nr)
    def pub(tt, carry):
        plsc.fetch_and_add(smem.at[sid], nr, subcore_id=tt)
        return carry
    lax.fori_loop(0, NS, pub, 0)
    plsc.subcore_barrier()
    off = jnp.int32(0)
    for t in range(NS):
        off = off + jnp.where(t < sid, smem[t], 0)

    @pl.when(cid == 0)
    def _emit():
        def cpk(k, carry):
            ko = pl.multiple_of(off + k * G, G)
            pltpu.sync_copy(dstage_i.at[pl.ds(pl.multiple_of(k * G, G), G)],
                            dupi_hbm.at[pl.ds(ko, G)])
            pltpu.sync_copy(dstage_y.at[pl.ds(pl.multiple_of(k * G, G), G)],
                            dupy_hbm.at[pl.ds(ko, G)])
            return carry
        lax.fori_loop(0, nr // G, cpk, 0)

    @pl.when((cid == 0) & (sid == 0))
    def _emit_cnts():
        for t in range(NS):
            cbuf[pl.ds(0, L)] = jnp.where(_iota() == t, smem[t],
                                          cbuf[pl.ds(0, L)])
        pltpu.sync_copy(cbuf, cnts_hbm)


def _k2_body(x_hbm, y_hbm, mu_hbm, ck_hbm, dupi_hbm, dupy_hbm, cnts_hbm, nmu,
             wini_sh, winy_sh,
             dstage_i, dstage_y, tag, wstage_i, wstage_y,
             wli, wly, muv2, xv2, ckv2, rcpv2, cbuf, smem, se):
    cid = lax.axis_index("c")
    sid = lax.axis_index("s")
    for i in range(18):
        smem[i] = 0
    plsc.subcore_barrier()

    @pl.when(sid == 0)
    def _resolve():
        pltpu.sync_copy(cnts_hbm, cbuf)
        ptot = _prefix_sum(cbuf[pl.ds(0, L)])[L - 1]
        nchunks = (ptot + RT - 1) // RT
        def p1(c, carry):
            nv = jnp.minimum(RT, ptot - c * RT) // L
            pltpu.sync_copy(dupy_hbm.at[pl.ds(pl.multiple_of(c * RT, RT), RT)],
                            dstage_y.at[pl.ds(0, RT)])
            def z1(v, carry2):
                yv = dstage_y[pl.ds(v * L, L)]
                plsc.store_scatter(tag, [yv], jnp.zeros((L,), _i32))
                return carry2
            lax.fori_loop(0, nv, z1, 0)
            return carry
        lax.fori_loop(0, nchunks, p1, 0)
        def p2(c, carry):
            nv = jnp.minimum(RT, ptot - c * RT) // L
            co = pl.multiple_of(c * RT, RT)
            pltpu.sync_copy(dupy_hbm.at[pl.ds(co, RT)],
                            dstage_y.at[pl.ds(0, RT)])
            pltpu.sync_copy(dupi_hbm.at[pl.ds(co, RT)],
                            dstage_i.at[pl.ds(0, RT)])
            def s1(v, carry2):
                yv = dstage_y[pl.ds(v * L, L)]
                iv = dstage_i[pl.ds(v * L, L)]
                key = yv * B + iv
                ks, _unused = plsc.sort_key_val(key, key)
                ys = lax.shift_right_logical(ks, 14)
                ii = lax.bitwise_and(ks, B - 1)
                nxt = _take16(ys, jnp.minimum(_iota() + 1, L - 1))
                keep = (ys != nxt) | (_iota() == (L - 1))
                plsc.store_scatter(tag, [ys], ii + 1, mask=keep)
                return carry2
            lax.fori_loop(0, nv, s1, 0)
            return carry
        lax.fori_loop(0, nchunks, p2, 0)
        def p3(c, wtot):
            nv = jnp.minimum(RT, ptot - c * RT) // L
            co = pl.multiple_of(c * RT, RT)
            pltpu.sync_copy(dupy_hbm.at[pl.ds(co, RT)],
                            dstage_y.at[pl.ds(0, RT)])
            pltpu.sync_copy(dupi_hbm.at[pl.ds(co, RT)],
                            dstage_i.at[pl.ds(0, RT)])
            def w1(v, woff):
                yv = dstage_y[pl.ds(v * L, L)]
                iv = dstage_i[pl.ds(v * L, L)]
                g = plsc.load_gather(tag, [yv])
                winm = (g == iv + 1) & ((iv % 2) == cid)
                return _compact2(wstage_i, wstage_y, woff, iv, yv, winm)
            nw = lax.fori_loop(0, nv, w1, jnp.int32(0))
            nwr = lax.bitwise_and(nw + (G - 1), ~(G - 1))
            _pad_replicate(wstage_i, wstage_y, nw, nwr)
            def wcp(k, carry2):
                ko = pl.multiple_of(wtot + k * G, G)
                pltpu.sync_copy(wstage_i.at[pl.ds(pl.multiple_of(k * G, G), G)],
                                wini_sh.at[pl.ds(ko, G)])
                pltpu.sync_copy(wstage_y.at[pl.ds(pl.multiple_of(k * G, G), G)],
                                winy_sh.at[pl.ds(ko, G)])
                return carry2
            lax.fori_loop(0, nwr // G, wcp, 0)
            return wtot + nwr
        nwin = lax.fori_loop(0, nchunks, p3, jnp.int32(0))
        def pubw(tt, carry):
            plsc.fetch_and_add(smem.at[16], nwin, subcore_id=tt)
            return carry
        lax.fori_loop(0, NS, pubw, 0)
    plsc.subcore_barrier()

    nwp = smem[16]
    nce = nwp // WCH
    trips = jnp.maximum((nce - sid + NS - 1) // NS, 0)
    def fix(m, carry):
        bw = pl.multiple_of((sid + m * NS) * WCH, WCH)
        d1 = pltpu.async_copy(wini_sh.at[pl.ds(bw, WCH)], wli.at[0], se)
        d2 = pltpu.async_copy(winy_sh.at[pl.ds(bw, WCH)], wly.at[0], se)
        d1.wait()
        d2.wait()
        d3 = pltpu.async_copy(mu_hbm.at[wly.at[0]], muv2, se)
        d4 = pltpu.async_copy(x_hbm.at[wli.at[0]], xv2, se)
        d5 = pltpu.async_copy(ck_hbm.at[wly.at[0]], ckv2, se)
        d3.wait()
        d4.wait()
        d5.wait()
        for v in range(WCH // L):
            sl = pl.ds(v * L, L)
            rcpv2[sl] = 1.0 / (ckv2[sl] + 1.0)
        _compute_rows(muv2, xv2, rcpv2, WCH)
        pltpu.sync_copy(muv2, nmu.at[wly.at[0]])
        return carry
    lax.fori_loop(0, trips, fix, 0)


def _k3_body(y_hbm, nmu, out_hbm, idx2, b0, b1, sidx, sg0, sg1, sw0, sw1):
    cid = lax.axis_index("c")
    sid = lax.axis_index("s")
    base = (sid * NC + cid) * RW
    dsc = [pltpu.async_copy(y_hbm.at[pl.ds(base + j * CH, CH)], idx2.at[j],
                            sidx) for j in range(NCH)]
    for d in dsc:
        d.wait()
    bufs, sg, sw = [b0, b1], [sg0, sg1], [sw0, sw1]
    pend = [None, None]
    gcur = pltpu.async_copy(nmu.at[idx2.at[0]], bufs[0], sg[0])
    for j in range(NCH):
        s = j % 2
        if j + 1 < NCH:
            if pend[1 - s] is not None:
                pend[1 - s].wait()
                pend[1 - s] = None
            gnxt = pltpu.async_copy(nmu.at[idx2.at[j + 1]], bufs[1 - s],
                                    sg[1 - s])
        gcur.wait()
        pend[s] = pltpu.async_copy(bufs[s],
                                   out_hbm.at[pl.ds(base + j * CH, CH)],
                                   sw[s])
        if j + 1 < NCH:
            gcur = gnxt
    for po in pend:
        if po is not None:
            po.wait()


_k1 = functools.partial(
    pl.kernel, mesh=_mesh, compiler_params=_cparams,
    out_type=(jax.ShapeDtypeStruct((B,), _i32),
              jax.ShapeDtypeStruct((B,), _i32),
              jax.ShapeDtypeStruct((L,), _i32)),
    scratch_types=[
        pltpu.VMEM_SHARED((CPAD,), _i32),
        pltpu.VMEM((NCH, CH), _i32),
        pltpu.VMEM((CH,), _f32),
        pltpu.VMEM((CH,), _f32),
        pltpu.VMEM((CH + L,), _f32),
        pltpu.VMEM((CH,), _f32),
        pltpu.VMEM((CH,), _f32),
        pltpu.VMEM((CH, D), _f32),
        pltpu.VMEM((CH, D), _f32),
        pltpu.VMEM((CH, D), _f32),
        pltpu.VMEM((CH, D), _f32),
        pltpu.VMEM((ZB,), _i32),
        pltpu.VMEM((RT // CH, CH), _i32),
        pltpu.VMEM((CH,), _i32),
        pltpu.VMEM((RT,), _i32),
        pltpu.VMEM((RT + G,), _i32),
        pltpu.VMEM((RT + G,), _i32),
        pltpu.VMEM((L,), _i32),
        pltpu.SMEM((32,), _i32),
        pltpu.SemaphoreType.DMA,
        pltpu.SemaphoreType.DMA,
        pltpu.SemaphoreType.DMA,
        pltpu.SemaphoreType.DMA,
        pltpu.SemaphoreType.DMA,
        pltpu.SemaphoreType.DMA,
        pltpu.SemaphoreType.DMA,
    ],
)(_k1_body)

_k2 = functools.partial(
    pl.kernel, mesh=_mesh, out_type=(), compiler_params=_cparams,
    scratch_types=[
        pltpu.VMEM_SHARED((WSH,), _i32),
        pltpu.VMEM_SHARED((WSH,), _i32),
        pltpu.VMEM((RT + G,), _i32),
        pltpu.VMEM((RT + G,), _i32),
        pltpu.VMEM((CPAD,), _i32),
        pltpu.VMEM((RT + G,), _i32),
        pltpu.VMEM((RT + G,), _i32),
        pltpu.VMEM((1, WCH), _i32),
        pltpu.VMEM((1, WCH), _i32),
        pltpu.VMEM((WCH, D), _f32),
        pltpu.VMEM((WCH, D), _f32),
        pltpu.VMEM((WCH,), _f32),
        pltpu.VMEM((WCH + L,), _f32),
        pltpu.VMEM((L,), _i32),
        pltpu.SMEM((32,), _i32),
        pltpu.SemaphoreType.DMA,
    ],
)(_k2_body)

_k3 = functools.partial(
    pl.kernel, mesh=_mesh, compiler_params=_cparams,
    out_type=jax.ShapeDtypeStruct((B, D), _f32),
    scratch_types=[
        pltpu.VMEM((NCH, CH), _i32),
        pltpu.VMEM((CH, D), _f32),
        pltpu.VMEM((CH, D), _f32),
        pltpu.SemaphoreType.DMA,
        pltpu.SemaphoreType.DMA,
        pltpu.SemaphoreType.DMA,
        pltpu.SemaphoreType.DMA,
        pltpu.SemaphoreType.DMA,
    ],
)(_k3_body)


def kernel(x, y, muK, cK):
    nmu = jax.new_ref(muK)
    nck = jax.new_ref(cK)
    dupi, dupy, cnts = _k1(x, y, muK, cK, nmu, nck)
    _k2(x, y, muK, cK, dupi, dupy, cnts, nmu)
    out = _k3(y, nmu)
    return out, nmu[...], nck[...]


# speedup vs baseline: 1.1040x; 1.0222x over previous
import functools

import jax
import jax.numpy as jnp
from jax import lax
from jax.experimental import pallas as pl
from jax.experimental.pallas import tpu as pltpu
from jax.experimental.pallas import tpu_sc as plsc

C = 100000
D = 128
B = 16384
L = 16
NC = 2
NS = 16
NW = NC * NS
RW = B // NW
CH = 128
NCH = RW // CH
CPAD = 100352
RT = B // NS
WCH = 64
G = 64
ZB = 784
WSH = B

_mesh = plsc.VectorSubcoreMesh(core_axis_name="c", subcore_axis_name="s",
                               num_cores=NC, num_subcores=NS)
_cparams = pltpu.CompilerParams(needs_layout_passes=False)

_i32 = jnp.int32
_f32 = jnp.float32


def _iota():
    return lax.iota(_i32, L)


def _take16(arr, idx):
    dnums = lax.GatherDimensionNumbers(
        offset_dims=(), collapsed_slice_dims=(0,), start_index_map=(0,))
    return lax.gather(arr, idx[:, None], dnums, slice_sizes=(1,),
                      mode=lax.GatherScatterMode.PROMISE_IN_BOUNDS)


def _bcast_row(ref, r):
    v = ref[pl.ds(r, L)]
    return _take16(v, jnp.zeros((L,), _i32))


def _prefix_sum(x):
    it = _iota()
    for s in (1, 2, 4, 8):
        sh = _take16(x, jnp.maximum(it - s, 0))
        x = x + jnp.where(it >= s, sh, 0)
    return x


def _compact2(ref_a, ref_b, off, vals_a, vals_b, mask):
    pref = _prefix_sum(mask.astype(_i32))
    pos = off + pref - 1
    plsc.store_scatter(ref_a, [pos], vals_a, mask=mask)
    plsc.store_scatter(ref_b, [pos], vals_b, mask=mask)
    return off + pref[L - 1]


def _pad_replicate(ref_a, ref_b, n, nr):
    last_a = _bcast_row(ref_a, jnp.maximum(n - 1, 0))
    last_b = _bcast_row(ref_b, jnp.maximum(n - 1, 0))
    def fill(p, carry):
        pos = p * L + _iota()
        m = (pos >= n) & (pos < nr)
        plsc.store_scatter(ref_a, [pos], last_a, mask=m)
        plsc.store_scatter(ref_b, [pos], last_b, mask=m)
        return carry
    lax.fori_loop(n // L, (nr + L - 1) // L, fill, 0)


def _compute_rows(muv, xv, rcpv, nrows):
    def row_body(r, carry):
        dv = _bcast_row(rcpv, r)
        for g in range(D // L):
            sl = pl.ds(g * L, L)
            m = muv[r, sl]
            xx = xv[r, sl]
            muv[r, sl] = m + (xx - m) * dv
        return carry
    lax.fori_loop(0, nrows, row_body, 0)


def _k1_body(x_hbm, y_hbm, mu_hbm, ck_hbm, nmu, nck,
             dupi_hbm, dupy_hbm, cnts_hbm,
             cnt_sh,
             idx2, ckv0, ckv1, rcpv, ckp0, ckp1, mu0, mu1, x0, x1,
             zbuf, idx3, onesb, cntv, dstage_i, dstage_y, cbuf, smem,
             sidx, sin0, sin1, sout0, sout1, sz, si):
    cid = lax.axis_index("c")
    sid = lax.axis_index("s")
    base = (sid * NC + cid) * RW

    for i in range(18):
        smem[i] = 0
    zslice = CPAD // NS
    def zb(i, carry):
        zbuf[pl.ds(i * L, L)] = jnp.zeros((L,), _i32)
        return carry
    lax.fori_loop(0, ZB // L, zb, 0)
    zd = [pltpu.async_copy(zbuf, cnt_sh.at[pl.ds(sid * zslice + q * ZB, ZB)],
                           sz) for q in range(zslice // ZB)]
    ld = [pltpu.async_copy(y_hbm.at[pl.ds(sid * RT + j * CH, CH)], idx3.at[j],
                           si) for j in range(RT // CH)]
    for v in range(CH // L):
        onesb[pl.ds(v * L, L)] = jnp.ones((L,), _i32)

    dsc = [pltpu.async_copy(y_hbm.at[pl.ds(base + j * CH, CH)], idx2.at[j],
                            sidx) for j in range(NCH)]
    for d in dsc:
        d.wait()
    mub, xb, ckb, ckpb = [mu0, mu1], [x0, x1], [ckv0, ckv1], [ckp0, ckp1]
    sin, sout = [sin0, sin1], [sout0, sout1]

    def fire(j):
        s = j % 2
        return (pltpu.async_copy(mu_hbm.at[idx2.at[j]], mub[s], sin[s]),
                pltpu.async_copy(x_hbm.at[pl.ds(base + j * CH, CH)], xb[s],
                                 sin[s]),
                pltpu.async_copy(ck_hbm.at[idx2.at[j]], ckb[s], sin[s]))

    pend = [None, None]
    ind = fire(0)
    for j in range(NCH):
        s = j % 2
        if j + 1 < NCH:
            if pend[1 - s] is not None:
                for d in pend[1 - s]:
                    d.wait()
                pend[1 - s] = None
            nxt = fire(j + 1)
        for d in ind:
            d.wait()
        for v in range(CH // L):
            sl = pl.ds(v * L, L)
            cc = ckb[s][sl] + 1.0
            ckpb[s][sl] = cc
            rcpv[sl] = 1.0 / cc
        _compute_rows(mub[s], xb[s], rcpv, CH)
        pend[s] = (pltpu.async_copy(mub[s], nmu.at[idx2.at[j]], sout[s]),
                   pltpu.async_copy(ckpb[s], nck.at[idx2.at[j]], sout[s]))
        if j + 1 < NCH:
            ind = nxt
    for po in pend:
        if po is not None:
            for d in po:
                d.wait()

    for d in zd + ld:
        d.wait()
    plsc.subcore_barrier()
    hd = [pltpu.async_copy(onesb, cnt_sh.at[idx3.at[j]], sz, add=True)
          for j in range(RT // CH)]
    for d in hd:
        d.wait()
    plsc.subcore_barrier()
    gd = [pltpu.async_copy(cnt_sh.at[idx3.at[j]],
                           cntv.at[pl.ds(j * CH, CH)], si)
          for j in range(RT // CH)]
    for d in gd:
        d.wait()
    ndup = jnp.int32(0)
    for v in range(RT // L):
        cv = cntv[pl.ds(v * L, L)]
        yv = idx3[v // (CH // L), pl.ds((v % (CH // L)) * L, L)]
        iv = _iota() + (sid * RT + v * L)
        ndup = _compact2(dstage_i, dstage_y, ndup, iv, yv, cv > 1)
    nr = lax.bitwise_and(ndup + (G - 1), ~(G - 1))
    _pad_replicate(dstage_i, dstage_y, ndup, nr)
    def pub(tt, carry):
        plsc.fetch_and_add(smem.at[sid], nr, subcore_id=tt)
        return carry
    lax.fori_loop(0, NS, pub, 0)
    plsc.subcore_barrier()
    off = jnp.int32(0)
    for t in range(NS):
        off = off + jnp.where(t < sid, smem[t], 0)

    @pl.when(cid == 0)
    def _emit():
        def cpk(k, carry):
            ko = pl.multiple_of(off + k * G, G)
            pltpu.sync_copy(dstage_i.at[pl.ds(pl.multiple_of(k * G, G), G)],
                            dupi_hbm.at[pl.ds(ko, G)])
            pltpu.sync_copy(dstage_y.at[pl.ds(pl.multiple_of(k * G, G), G)],
                            dupy_hbm.at[pl.ds(ko, G)])
            return carry
        lax.fori_loop(0, nr // G, cpk, 0)

    @pl.when((cid == 0) & (sid == 0))
    def _emit_cnts():
        for t in range(NS):
            cbuf[pl.ds(0, L)] = jnp.where(_iota() == t, smem[t],
                                          cbuf[pl.ds(0, L)])
        pltpu.sync_copy(cbuf, cnts_hbm)


def _k2_body(x_hbm, y_hbm, mu_hbm, ck_hbm, dupi_hbm, dupy_hbm, cnts_hbm, nmu,
             wini_sh, winy_sh,
             dstage_i, dstage_y, tag, wstage_i, wstage_y,
             wli, wly, muv2, xv2, ckv2, rcpv2, cbuf, smem, se):
    cid = lax.axis_index("c")
    sid = lax.axis_index("s")
    for i in range(18):
        smem[i] = 0
    plsc.subcore_barrier()

    @pl.when(sid == 0)
    def _resolve():
        pltpu.sync_copy(cnts_hbm, cbuf)
        ptot = _prefix_sum(cbuf[pl.ds(0, L)])[L - 1]
        nchunks = (ptot + RT - 1) // RT
        def p1(c, carry):
            nv = jnp.minimum(RT, ptot - c * RT) // L
            pltpu.sync_copy(dupy_hbm.at[pl.ds(pl.multiple_of(c * RT, RT), RT)],
                            dstage_y.at[pl.ds(0, RT)])
            def z1(v, carry2):
                yv = dstage_y[pl.ds(v * L, L)]
                plsc.store_scatter(tag, [yv], jnp.zeros((L,), _i32))
                return carry2
            lax.fori_loop(0, nv, z1, 0)
            return carry
        lax.fori_loop(0, nchunks, p1, 0)
        def p2(c, carry):
            nv = jnp.minimum(RT, ptot - c * RT) // L
            co = pl.multiple_of(c * RT, RT)
            da = pltpu.async_copy(dupy_hbm.at[pl.ds(co, RT)],
                                  dstage_y.at[pl.ds(0, RT)], se)
            db = pltpu.async_copy(dupi_hbm.at[pl.ds(co, RT)],
                                  dstage_i.at[pl.ds(0, RT)], se)
            da.wait()
            db.wait()
            def s1(v, carry2):
                yv = dstage_y[pl.ds(v * L, L)]
                iv = dstage_i[pl.ds(v * L, L)]
                key = yv * B + iv
                ks, _unused = plsc.sort_key_val(key, key)
                ys = lax.shift_right_logical(ks, 14)
                ii = lax.bitwise_and(ks, B - 1)
                nxt = _take16(ys, jnp.minimum(_iota() + 1, L - 1))
                keep = (ys != nxt) | (_iota() == (L - 1))
                plsc.store_scatter(tag, [ys], ii + 1, mask=keep)
                return carry2
            lax.fori_loop(0, nv, s1, 0)
            return carry
        lax.fori_loop(0, nchunks, p2, 0)
        def p3(c, wtot):
            nv = jnp.minimum(RT, ptot - c * RT) // L
            co = pl.multiple_of(c * RT, RT)
            da = pltpu.async_copy(dupy_hbm.at[pl.ds(co, RT)],
                                  dstage_y.at[pl.ds(0, RT)], se)
            db = pltpu.async_copy(dupi_hbm.at[pl.ds(co, RT)],
                                  dstage_i.at[pl.ds(0, RT)], se)
            da.wait()
            db.wait()
            def w1(v, woff):
                yv = dstage_y[pl.ds(v * L, L)]
                iv = dstage_i[pl.ds(v * L, L)]
                g = plsc.load_gather(tag, [yv])
                winm = (g == iv + 1) & ((iv % 2) == cid)
                return _compact2(wstage_i, wstage_y, woff, iv, yv, winm)
            nw = lax.fori_loop(0, nv, w1, jnp.int32(0))
            nwr = lax.bitwise_and(nw + (G - 1), ~(G - 1))
            _pad_replicate(wstage_i, wstage_y, nw, nwr)
            def wcp(k, carry2):
                ko = pl.multiple_of(wtot + k * G, G)
                pltpu.sync_copy(wstage_i.at[pl.ds(pl.multiple_of(k * G, G), G)],
                                wini_sh.at[pl.ds(ko, G)])
                pltpu.sync_copy(wstage_y.at[pl.ds(pl.multiple_of(k * G, G), G)],
                                winy_sh.at[pl.ds(ko, G)])
                return carry2
            lax.fori_loop(0, nwr // G, wcp, 0)
            return wtot + nwr
        nwin = lax.fori_loop(0, nchunks, p3, jnp.int32(0))
        def pubw(tt, carry):
            plsc.fetch_and_add(smem.at[16], nwin, subcore_id=tt)
            return carry
        lax.fori_loop(0, NS, pubw, 0)
    plsc.subcore_barrier()

    nwp = smem[16]
    nce = nwp // WCH
    trips = jnp.maximum((nce - sid + NS - 1) // NS, 0)
    def fix(m, carry):
        bw = pl.multiple_of((sid + m * NS) * WCH, WCH)
        d1 = pltpu.async_copy(wini_sh.at[pl.ds(bw, WCH)], wli.at[0], se)
        d2 = pltpu.async_copy(winy_sh.at[pl.ds(bw, WCH)], wly.at[0], se)
        d1.wait()
        d2.wait()
        d3 = pltpu.async_copy(mu_hbm.at[wly.at[0]], muv2, se)
        d4 = pltpu.async_copy(x_hbm.at[wli.at[0]], xv2, se)
        d5 = pltpu.async_copy(ck_hbm.at[wly.at[0]], ckv2, se)
        d3.wait()
        d4.wait()
        d5.wait()
        for v in range(WCH // L):
            sl = pl.ds(v * L, L)
            rcpv2[sl] = 1.0 / (ckv2[sl] + 1.0)
        _compute_rows(muv2, xv2, rcpv2, WCH)
        pltpu.sync_copy(muv2, nmu.at[wly.at[0]])
        return carry
    lax.fori_loop(0, trips, fix, 0)


def _k3_body(y_hbm, nmu, out_hbm, idx2, b0, b1, sidx, sg0, sg1, sw0, sw1):
    cid = lax.axis_index("c")
    sid = lax.axis_index("s")
    base = (sid * NC + cid) * RW
    dsc = [pltpu.async_copy(y_hbm.at[pl.ds(base + j * CH, CH)], idx2.at[j],
                            sidx) for j in range(NCH)]
    for d in dsc:
        d.wait()
    bufs, sg, sw = [b0, b1], [sg0, sg1], [sw0, sw1]
    pend = [None, None]
    gcur = pltpu.async_copy(nmu.at[idx2.at[0]], bufs[0], sg[0])
    for j in range(NCH):
        s = j % 2
        if j + 1 < NCH:
            if pend[1 - s] is not None:
                pend[1 - s].wait()
                pend[1 - s] = None
            gnxt = pltpu.async_copy(nmu.at[idx2.at[j + 1]], bufs[1 - s],
                                    sg[1 - s])
        gcur.wait()
        pend[s] = pltpu.async_copy(bufs[s],
                                   out_hbm.at[pl.ds(base + j * CH, CH)],
                                   sw[s])
        if j + 1 < NCH:
            gcur = gnxt
    for po in pend:
        if po is not None:
            po.wait()


_k1 = functools.partial(
    pl.kernel, mesh=_mesh, compiler_params=_cparams,
    out_type=(jax.ShapeDtypeStruct((B,), _i32),
              jax.ShapeDtypeStruct((B,), _i32),
              jax.ShapeDtypeStruct((L,), _i32)),
    scratch_types=[
        pltpu.VMEM_SHARED((CPAD,), _i32),
        pltpu.VMEM((NCH, CH), _i32),
        pltpu.VMEM((CH,), _f32),
        pltpu.VMEM((CH,), _f32),
        pltpu.VMEM((CH + L,), _f32),
        pltpu.VMEM((CH,), _f32),
        pltpu.VMEM((CH,), _f32),
        pltpu.VMEM((CH, D), _f32),
        pltpu.VMEM((CH, D), _f32),
        pltpu.VMEM((CH, D), _f32),
        pltpu.VMEM((CH, D), _f32),
        pltpu.VMEM((ZB,), _i32),
        pltpu.VMEM((RT // CH, CH), _i32),
        pltpu.VMEM((CH,), _i32),
        pltpu.VMEM((RT,), _i32),
        pltpu.VMEM((RT + G,), _i32),
        pltpu.VMEM((RT + G,), _i32),
        pltpu.VMEM((L,), _i32),
        pltpu.SMEM((32,), _i32),
        pltpu.SemaphoreType.DMA,
        pltpu.SemaphoreType.DMA,
        pltpu.SemaphoreType.DMA,
        pltpu.SemaphoreType.DMA,
        pltpu.SemaphoreType.DMA,
        pltpu.SemaphoreType.DMA,
        pltpu.SemaphoreType.DMA,
    ],
)(_k1_body)

_k2 = functools.partial(
    pl.kernel, mesh=_mesh, out_type=(), compiler_params=_cparams,
    scratch_types=[
        pltpu.VMEM_SHARED((WSH,), _i32),
        pltpu.VMEM_SHARED((WSH,), _i32),
        pltpu.VMEM((RT + G,), _i32),
        pltpu.VMEM((RT + G,), _i32),
        pltpu.VMEM((CPAD,), _i32),
        pltpu.VMEM((RT + G,), _i32),
        pltpu.VMEM((RT + G,), _i32),
        pltpu.VMEM((1, WCH), _i32),
        pltpu.VMEM((1, WCH), _i32),
        pltpu.VMEM((WCH, D), _f32),
        pltpu.VMEM((WCH, D), _f32),
        pltpu.VMEM((WCH,), _f32),
        pltpu.VMEM((WCH + L,), _f32),
        pltpu.VMEM((L,), _i32),
        pltpu.SMEM((32,), _i32),
        pltpu.SemaphoreType.DMA,
    ],
)(_k2_body)

_k3 = functools.partial(
    pl.kernel, mesh=_mesh, compiler_params=_cparams,
    out_type=jax.ShapeDtypeStruct((B, D), _f32),
    scratch_types=[
        pltpu.VMEM((NCH, CH), _i32),
        pltpu.VMEM((CH, D), _f32),
        pltpu.VMEM((CH, D), _f32),
        pltpu.SemaphoreType.DMA,
        pltpu.SemaphoreType.DMA,
        pltpu.SemaphoreType.DMA,
        pltpu.SemaphoreType.DMA,
        pltpu.SemaphoreType.DMA,
    ],
)(_k3_body)


def kernel(x, y, muK, cK):
    nmu = jax.new_ref(muK)
    nck = jax.new_ref(cK)
    dupi, dupy, cnts = _k1(x, y, muK, cK, nmu, nck)
    _k2(x, y, muK, cK, dupi, dupy, cnts, nmu)
    out = _k3(y, nmu)
    return out, nmu[...], nck[...]
